# Initial kernel scaffold; baseline (speedup 1.0000x reference)
import jax
import jax.numpy as jnp
from jax.experimental import pallas as pl


def kernel(x, edge_attr, edge_index, nW1, nb1, nW2, nb2, nlg, nlb, eW1, eb1, eW2, eb2, elg, elb, beW1, beb1, beW2, beb2, belg, belb, bnW1, bnb1, bnW2, bnb2, bnlg, bnlb, dW1, db1, dW2, db2):
    raise NotImplementedError("write your pallas kernel here")


# speedup vs baseline: 3.2215x; 3.2215x over previous
import jax
import jax.numpy as jnp
from jax import lax
from jax.experimental import pallas as pl
from jax.experimental.pallas import tpu as pltpu
from jax.experimental.pallas import tpu_sc as plsc

N = 10000
E = 320000
H = 64
STEPS = 8
OUT = 3

IDXR, IDXC = 2500, 128
GW = 2
EB = 2000
SN = N // 16

_mesh = plsc.VectorSubcoreMesh(core_axis_name="core", subcore_axis_name="subcore")
_sc_params = pltpu.CompilerParams(use_tc_tiling_on_sc=False)


def _sc_gather2(P, Q, snd2, rcv2):
    out_t = (jax.ShapeDtypeStruct((E, H), jnp.float32),
             jax.ShapeDtypeStruct((E, H), jnp.float32))

    @pl.kernel(out_type=out_t, mesh=_mesh, compiler_params=_sc_params)
    def k(p_hbm, q_hbm, si_hbm, ri_hbm, rs_hbm, rr_hbm):
        def body(si, ri, rs, rr):
            for j in range(GW):
                pltpu.sync_copy(p_hbm.at[si.at[j]], rs.at[pl.ds(j * IDXC, IDXC)])
                pltpu.sync_copy(q_hbm.at[ri.at[j]], rr.at[pl.ds(j * IDXC, IDXC)])

        pltpu.emit_pipeline(
            body,
            grid=(IDXR // GW,),
            in_specs=[pl.BlockSpec((GW, IDXC), lambda i: (i, 0)),
                      pl.BlockSpec((GW, IDXC), lambda i: (i, 0))],
            out_specs=[pl.BlockSpec((GW * IDXC, H), lambda i: (i, 0)),
                       pl.BlockSpec((GW * IDXC, H), lambda i: (i, 0))],
            core_axis_name=("core", "subcore"),
            dimension_semantics=(pltpu.PARALLEL,),
        )(si_hbm, ri_hbm, rs_hbm, rr_hbm)

    return k(P, Q, snd2, rcv2)


def _sc_scatter_add(e_upd, rcv2, zrows):
    @pl.kernel(out_type=jax.ShapeDtypeStruct((2, N, H), jnp.float32),
               mesh=_mesh, compiler_params=_sc_params,
               scratch_types=[pltpu.VMEM_SHARED((N, H), jnp.float32)])
    def k(x_hbm, i_hbm, z_hbm, o_hbm, acc):
        cid = lax.axis_index("core")
        sid = lax.axis_index("subcore")
        r0 = sid * SN
        pltpu.sync_copy(z_hbm, acc.at[pl.ds(r0, SN)])
        plsc.subcore_barrier()

        def body(x, i):
            for j in range(GW):
                pltpu.sync_copy(x.at[pl.ds(j * IDXC, IDXC)], acc.at[i.at[j]],
                                add=True)

        pltpu.emit_pipeline(
            body,
            grid=(IDXR // GW,),
            in_specs=[pl.BlockSpec((GW * IDXC, H), lambda i: (i, 0)),
                      pl.BlockSpec((GW, IDXC), lambda i: (i, 0))],
            out_specs=[],
            core_axis_name=("core", "subcore"),
            dimension_semantics=(pltpu.PARALLEL,),
        )(x_hbm, i_hbm)

        plsc.subcore_barrier()
        pltpu.sync_copy(acc.at[pl.ds(r0, SN)], o_hbm.at[cid, pl.ds(r0, SN)])

    return k(e_upd, rcv2, zrows)


def _ln(h, g, b):
    mu = jnp.mean(h, axis=-1, keepdims=True)
    var = jnp.mean((h - mu) ** 2, axis=-1, keepdims=True)
    return (h - mu) / jnp.sqrt(var + 1e-5) * g + b


def _dot(a, b):
    return jnp.dot(a, b, preferred_element_type=jnp.float32)


def _node_encoder(x, nW1, nb1, nW2, nb2, nlg, nlb, Ws, Wr):
    def body(x_r, w1, b1, w2, b2, g, b, ws, wr, nfo, po, qo):
        h = jnp.maximum(_dot(x_r[...], w1[...]) + b1[...], 0.0)
        nf = _ln(_dot(h, w2[...]) + b2[...], g[...], b[...])
        nfo[...] = nf
        po[...] = _dot(nf, ws[...])
        qo[...] = _dot(nf, wr[...])

    return pl.pallas_call(
        body,
        out_shape=[jax.ShapeDtypeStruct((N, H), jnp.float32)] * 3,
    )(x, nW1, nb1, nW2, nb2, nlg, nlb, Ws, Wr)


def _edge_encoder(edge_attr, eW1, eb1, eW2, eb2, elg, elb):
    def body(a_r, w1, b1, w2, b2, g, b, efo):
        h = jnp.maximum(_dot(a_r[...], w1[...]) + b1[...], 0.0)
        efo[...] = _ln(_dot(h, w2[...]) + b2[...], g[...], b[...])

    D = edge_attr.shape[1]
    wspec = lambda a, b: pl.BlockSpec((a, b), lambda i: (0, 0))
    return pl.pallas_call(
        body,
        grid=(E // EB,),
        in_specs=[pl.BlockSpec((EB, D), lambda i: (i, 0)),
                  wspec(D, H), wspec(1, H), wspec(H, H), wspec(1, H),
                  wspec(1, H), wspec(1, H)],
        out_specs=pl.BlockSpec((EB, H), lambda i: (i, 0)),
        out_shape=jax.ShapeDtypeStruct((E, H), jnp.float32),
    )(edge_attr, eW1, eb1, eW2, eb2, elg, elb)


def _edge_mlp(Rs, Rr, ef, We, b1, W2, b2, g, b):
    def body(rs, rr, ef_r, we, b1r, w2, b2r, gr, br, eu, efn):
        h = jnp.maximum(rs[...] + rr[...] + _dot(ef_r[...], we[...]) + b1r[...],
                        0.0)
        e = _ln(_dot(h, w2[...]) + b2r[...], gr[...], br[...])
        eu[...] = e
        efn[...] = ef_r[...] + e

    blk = lambda: pl.BlockSpec((EB, H), lambda i: (i, 0))
    wspec = lambda a, b: pl.BlockSpec((a, b), lambda i: (0, 0))
    return pl.pallas_call(
        body,
        grid=(E // EB,),
        in_specs=[blk(), blk(), blk(), wspec(H, H), wspec(1, H), wspec(H, H),
                  wspec(1, H), wspec(1, H), wspec(1, H)],
        out_specs=[blk(), blk()],
        out_shape=[jax.ShapeDtypeStruct((E, H), jnp.float32)] * 2,
    )(Rs, Rr, ef, We, b1, W2, b2, g, b)


def _node_mlp(nf, aggs, Wn, Wa, b1, W2, b2, g, b, Ws, Wr):
    def body(nf_r, ag, wn, wa, b1r, w2, b2r, gr, br, ws, wr, nfo, po, qo):
        agg = ag[0] + ag[1]
        h = jnp.maximum(_dot(nf_r[...], wn[...]) + _dot(agg, wa[...]) + b1r[...],
                        0.0)
        nu = _ln(_dot(h, w2[...]) + b2r[...], gr[...], br[...])
        nfn = nf_r[...] + nu
        nfo[...] = nfn
        po[...] = _dot(nfn, ws[...])
        qo[...] = _dot(nfn, wr[...])

    return pl.pallas_call(
        body,
        out_shape=[jax.ShapeDtypeStruct((N, H), jnp.float32)] * 3,
    )(nf, aggs, Wn, Wa, b1, W2, b2, g, b, Ws, Wr)


def _node_mlp_last(nf, aggs, Wn, Wa, b1, W2, b2, g, b, dW1, db1, dW2, db2):
    def body(nf_r, ag, wn, wa, b1r, w2, b2r, gr, br, w1d, b1d, w2d, b2d, o):
        agg = ag[0] + ag[1]
        h = jnp.maximum(_dot(nf_r[...], wn[...]) + _dot(agg, wa[...]) + b1r[...],
                        0.0)
        nu = _ln(_dot(h, w2[...]) + b2r[...], gr[...], br[...])
        nfn = nf_r[...] + nu
        hd = jnp.maximum(_dot(nfn, w1d[...]) + b1d[...], 0.0)
        o[...] = _dot(hd, w2d[...]) + b2d[...]

    return pl.pallas_call(
        body,
        out_shape=jax.ShapeDtypeStruct((N, OUT), jnp.float32),
    )(nf, aggs, Wn, Wa, b1, W2, b2, g, b, dW1, db1, dW2, db2)


def kernel(x, edge_attr, edge_index, nW1, nb1, nW2, nb2, nlg, nlb,
           eW1, eb1, eW2, eb2, elg, elb,
           beW1, beb1, beW2, beb2, belg, belb,
           bnW1, bnb1, bnW2, bnb2, bnlg, bnlb,
           dW1, db1, dW2, db2):
    r1 = lambda v: v.reshape(1, -1)
    snd2 = edge_index[0].reshape(IDXR, IDXC)
    rcv2 = edge_index[1].reshape(IDXR, IDXC)
    zrows = jnp.zeros((SN, H), jnp.float32)

    nf, P, Q = _node_encoder(x, nW1, r1(nb1), nW2, r1(nb2), r1(nlg), r1(nlb),
                             beW1[0, :H], beW1[0, H:2 * H])
    ef = _edge_encoder(edge_attr, eW1, r1(eb1), eW2, r1(eb2), r1(elg), r1(elb))

    for i in range(STEPS):
        Rs, Rr = _sc_gather2(P, Q, snd2, rcv2)
        e_upd, ef = _edge_mlp(Rs, Rr, ef, beW1[i, 2 * H:], r1(beb1[i]),
                              beW2[i], r1(beb2[i]), r1(belg[i]), r1(belb[i]))
        aggs = _sc_scatter_add(e_upd, rcv2, zrows)
        if i < STEPS - 1:
            nf, P, Q = _node_mlp(nf, aggs, bnW1[i, :H], bnW1[i, H:],
                                 r1(bnb1[i]), bnW2[i], r1(bnb2[i]),
                                 r1(bnlg[i]), r1(bnlb[i]),
                                 beW1[i + 1, :H], beW1[i + 1, H:2 * H])
        else:
            out = _node_mlp_last(nf, aggs, bnW1[i, :H], bnW1[i, H:],
                                 r1(bnb1[i]), bnW2[i], r1(bnb2[i]),
                                 r1(bnlg[i]), r1(bnlb[i]),
                                 dW1, r1(db1), dW2, r1(db2))
    return out
